# Initial kernel scaffold; baseline (speedup 1.0000x reference)
import jax
import jax.numpy as jnp
from jax.experimental import pallas as pl


def kernel(x, edge_index, W1, b1, W2, b2, W3, b3):
    raise NotImplementedError("write your pallas kernel here")


# speedup vs baseline: 2.8923x; 2.8923x over previous
import dataclasses
import functools

import jax
import jax.numpy as jnp
from jax import lax
from jax.experimental import pallas as pl
from jax.experimental.pallas import tpu as pltpu
from jax.experimental.pallas import tpu_sc as plsc

N = 10000
D = 128
E = 320000
NP = 10240
EP = 327680
NTILES = 32
EPT = EP // NTILES
C = 128
CHUNKS = EPT // C
RPS = NP // 16
DEGW = 16

_MESH = plsc.VectorSubcoreMesh(core_axis_name="c", subcore_axis_name="s")


@functools.partial(
    pl.kernel,
    out_type=jax.ShapeDtypeStruct((2, NP, D), jnp.float32),
    mesh=_MESH,
    scratch_types=[
        pltpu.VMEM((C,), jnp.int32),
        pltpu.VMEM((C,), jnp.int32),
        pltpu.VMEM((C, D), jnp.float32),
        pltpu.VMEM_SHARED((NP, D), jnp.float32),
        pltpu.SemaphoreType.DMA,
    ])
def _segsum(x_hbm, src_hbm, dst_hbm, out_hbm, sidx, didx, rows, acc, sem):
  cid = lax.axis_index("c")
  sid = lax.axis_index("s")

  @pl.loop(0, C)
  def _(i):
    for c in range(D // 16):
      rows.at[pl.ds(i, 1), pl.ds(c * 16, 16)][...] = jnp.zeros(
          (1, 16), jnp.float32)

  for k in range(RPS // C):
    r0 = sid * RPS + k * C
    pltpu.sync_copy(rows, acc.at[pl.ds(r0, C)])
  plsc.subcore_barrier()

  ebase = (cid * 16 + sid) * EPT

  @pl.loop(0, CHUNKS)
  def _(j):
    b0 = ebase + j * C
    pltpu.sync_copy(src_hbm.at[pl.ds(b0, C)], sidx)
    pltpu.sync_copy(dst_hbm.at[pl.ds(b0, C)], didx)
    pltpu.async_copy(x_hbm.at[sidx], rows, sem).wait()
    pltpu.sync_copy(rows, acc.at[didx], add=True)

  plsc.subcore_barrier()
  for k in range(RPS // C):
    r0 = sid * RPS + k * C
    pltpu.sync_copy(acc.at[pl.ds(r0, C)], out_hbm.at[cid].at[pl.ds(r0, C)])


_CP = pltpu.CompilerParams()
if "needs_layout_passes" in pltpu.CompilerParams.__dataclass_fields__:
  _CP = dataclasses.replace(_CP, needs_layout_passes=False)


@functools.partial(
    pl.kernel,
    out_type=jax.ShapeDtypeStruct((NTILES, NP), jnp.float32),
    mesh=_MESH,
    compiler_params=_CP,
    scratch_types=[
        pltpu.VMEM((EPT,), jnp.int32),
        pltpu.VMEM((NP,), jnp.float32),
    ])
def _deghist(dst_hbm, out_hbm, didx, hist):
  cid = lax.axis_index("c")
  sid = lax.axis_index("s")
  wid = cid * 16 + sid

  @pl.loop(0, NP // 16)
  def _(i):
    hist.at[pl.ds(i * 16, 16)][...] = jnp.zeros((16,), jnp.float32)

  pltpu.sync_copy(dst_hbm.at[pl.ds(wid * EPT, EPT)], didx)
  ones16 = jnp.ones((16,), jnp.float32)

  @pl.loop(0, EPT // 16)
  def _(k):
    idx = didx[pl.ds(k * 16, 16)]
    plsc.addupdate_scatter(hist, [idx], ones16)

  pltpu.sync_copy(hist, out_hbm.at[wid])


def _make_tc(with_relu: bool, final: bool):
  def body(p_ref, dg_ref, w_ref, b_ref, *rest):
    if final:
      w3_ref, b3_ref, o_ref = rest
    else:
      (o_ref,) = rest
    g = p_ref[0] + p_ref[1]
    deg = jnp.sum(dg_ref[...], axis=0)[:, None]
    xx = jnp.dot(g, w_ref[...], preferred_element_type=jnp.float32,
                 precision=lax.Precision.HIGHEST)
    xx = xx + deg * b_ref[...]
    if with_relu:
      xx = jnp.maximum(xx, 0.0)
    if final:
      xx = jnp.dot(xx, w3_ref[...], preferred_element_type=jnp.float32,
                   precision=lax.Precision.HIGHEST) + b3_ref[...]
    o_ref[...] = xx

  return pl.pallas_call(
      body, out_shape=jax.ShapeDtypeStruct((NP, D), jnp.float32))


_tc_step = _make_tc(False, False)
_tc_step_relu = _make_tc(True, False)
_tc_final = _make_tc(True, True)


def kernel(x, edge_index, W1, b1, W2, b2, W3, b3):
  src = edge_index[0].astype(jnp.int32)
  dst = edge_index[1].astype(jnp.int32)
  pad = EP - E
  src = jnp.concatenate([src, jnp.zeros((pad,), jnp.int32)])
  dst = jnp.concatenate([dst, jnp.full((pad,), NP - 1, jnp.int32)])
  xp = jnp.pad(x, ((0, NP - N), (0, 0)))
  b1r, b2r, b3r = b1.reshape(1, D), b2.reshape(1, D), b3.reshape(1, D)

  degp = _deghist(dst)
  P = _segsum(xp, src, dst)
  x1 = _tc_step(P, degp, W1, b1r)
  P = _segsum(x1, src, dst)
  x2 = _tc_step_relu(P, degp, W1, b1r)
  P = _segsum(x2, src, dst)
  x3 = _tc_step(P, degp, W2, b2r)
  P = _segsum(x3, src, dst)
  out = _tc_final(P, degp, W2, b2r, W3, b3r)
  return out[:N]
